# baseline (speedup 1.0000x reference)
import functools

import jax
import jax.numpy as jnp
from jax import lax
from jax.experimental import pallas as pl
from jax.experimental.pallas import tpu as pltpu
from jax.experimental.pallas import tpu_sc as plsc

_D = 64
_B = 16384
_NC = 2
_NS = 16
_NW = _NC * _NS
_BPW = _B // _NW
_CHUNK = 128
_NCHUNK = _BPW // _CHUNK
_LANES = _D // 16
_SCALE = 1.0 / float(_B * _D)


def _sc_body(features_hbm, labels_hbm, centers_hbm, out_hbm,
             idx_v, rows_v, feat_v, part_v, shared_v, gath_v, sem):
    cid = lax.axis_index("c")
    sid = lax.axis_index("s")
    wid = sid * _NC + cid
    base = wid * _BPW

    pltpu.sync_copy(labels_hbm.at[wid], idx_v)

    feat_copy = pltpu.async_copy(
        features_hbm.at[pl.ds(base, _BPW)], feat_v, sem)
    gathers = []
    for k in range(_NCHUNK):
        gathers.append(pltpu.async_copy(
            centers_hbm.at[idx_v.at[k]],
            rows_v.at[pl.ds(k * _CHUNK, _CHUNK)], sem))
    feat_copy.wait()
    for g in gathers:
        g.wait()

    def row_body(r, acc):
        for c in range(_LANES):
            f = feat_v[r, pl.ds(c * 16, 16)]
            t = rows_v[r, pl.ds(c * 16, 16)]
            d = f - t
            acc = acc + d * d
        return acc

    acc = lax.fori_loop(0, _BPW, row_body, jnp.zeros((16,), jnp.float32))

    part_v[...] = acc
    pltpu.sync_copy(part_v, shared_v.at[sid])
    plsc.subcore_barrier()

    @pl.when(sid == 0)
    def _():
        pltpu.sync_copy(shared_v, gath_v)
        tot = gath_v[0, :]
        for s in range(1, _NS):
            tot = tot + gath_v[s, :]
        part_v[...] = tot * _SCALE
        pltpu.sync_copy(part_v, out_hbm.at[cid])


@functools.partial(
    pl.kernel,
    out_type=jax.ShapeDtypeStruct((_NC, 16), jnp.float32),
    mesh=plsc.VectorSubcoreMesh(core_axis_name="c", subcore_axis_name="s"),
    compiler_params=pltpu.CompilerParams(use_tc_tiling_on_sc=False),
    scratch_types=[
        pltpu.VMEM((_NCHUNK, _CHUNK), jnp.int32),
        pltpu.VMEM((_BPW, _D), jnp.float32),
        pltpu.VMEM((_BPW, _D), jnp.float32),
        pltpu.VMEM((16,), jnp.float32),
        pltpu.VMEM_SHARED((_NS, 16), jnp.float32),
        pltpu.VMEM((_NS, 16), jnp.float32),
        pltpu.SemaphoreType.DMA,
    ],
)
def _center_loss_sc(features_hbm, labels_hbm, centers_hbm, out_hbm,
                    idx_v, rows_v, feat_v, part_v, shared_v, gath_v, sem):
    _sc_body(features_hbm, labels_hbm, centers_hbm, out_hbm,
             idx_v, rows_v, feat_v, part_v, shared_v, gath_v, sem)


@jax.jit
def kernel(features, labels, centers):
    labels_r = labels.reshape(_NW, _NCHUNK, _CHUNK)
    partials = _center_loss_sc(features, labels_r, centers)
    return jnp.sum(partials)
